# baseline (speedup 1.0000x reference)
import functools

import jax
import jax.numpy as jnp
from jax import lax
from jax.experimental import pallas as pl
from jax.experimental.pallas import tpu as pltpu
from jax.experimental.pallas import tpu_sc as plsc

N_TOKENS = 32768
N_EMBED = 768
NUM_EXPERTS = 8
TOP_K = 2

BT = 4096

NC, NS, L = 2, 16, 16
NW = NC * NS


def _logits_kernel(x_ref, wt_ref, b_ref, lt_ref):
    logits = jax.lax.dot_general(
        x_ref[...], wt_ref[...],
        dimension_numbers=(((1,), (0,)), ((), ())),
        preferred_element_type=jnp.float32,
    )
    lt_ref[...] = logits.T + b_ref[...]


def _tc_logits(x, wt, b2):
    n_tokens = x.shape[0]
    return pl.pallas_call(
        _logits_kernel,
        grid=(n_tokens // BT,),
        in_specs=[
            pl.BlockSpec((BT, N_EMBED), lambda i: (i, 0)),
            pl.BlockSpec((N_EMBED, NUM_EXPERTS), lambda i: (0, 0)),
            pl.BlockSpec((NUM_EXPERTS, 1), lambda i: (0, 0)),
        ],
        out_specs=pl.BlockSpec((NUM_EXPERTS, BT), lambda i: (0, i)),
        out_shape=jax.ShapeDtypeStruct((NUM_EXPERTS, n_tokens), jnp.float32),
    )(x, wt, b2)


def _sc_route(lt):
    n = lt.shape[1]
    bpw = n // NW
    mesh = plsc.VectorSubcoreMesh(core_axis_name="c", subcore_axis_name="s")

    @functools.partial(
        pl.kernel,
        mesh=mesh,
        out_type=[
            jax.ShapeDtypeStruct((NUM_EXPERTS, n), jnp.float32),
            jax.ShapeDtypeStruct((TOP_K, n), jnp.int32),
        ],
        scratch_types=(
            [pltpu.VMEM((bpw,), jnp.float32) for _ in range(NUM_EXPERTS)]
            + [pltpu.VMEM((bpw,), jnp.float32) for _ in range(NUM_EXPERTS)]
            + [pltpu.VMEM((bpw,), jnp.int32) for _ in range(TOP_K)]
        ),
    )
    def route(lt_hbm, outt_hbm, idxt_hbm, *scratch):
        l_refs = scratch[:NUM_EXPERTS]
        o_refs = scratch[NUM_EXPERTS:2 * NUM_EXPERTS]
        i1_ref, i2_ref = scratch[2 * NUM_EXPERTS:]
        wid = lax.axis_index("s") * NC + lax.axis_index("c")
        base = wid * bpw
        for e in range(NUM_EXPERTS):
            pltpu.sync_copy(lt_hbm.at[e, pl.ds(base, bpw)], l_refs[e])

        @pl.loop(0, bpw, step=L)
        def _(c):
            sl = pl.ds(c, L)
            lv = [l_refs[e][sl] for e in range(NUM_EXPERTS)]
            m1 = lv[0]
            i1 = jnp.zeros((L,), jnp.int32)
            m2 = jnp.full((L,), -jnp.inf, jnp.float32)
            i2 = jnp.zeros((L,), jnp.int32)
            for e in range(1, NUM_EXPERTS):
                gt1 = lv[e] > m1
                gt2 = lv[e] > m2
                nm2 = jnp.where(gt1, m1, jnp.where(gt2, lv[e], m2))
                ni2 = jnp.where(gt1, i1, jnp.where(gt2, e, i2))
                m1 = jnp.where(gt1, lv[e], m1)
                i1 = jnp.where(gt1, e, i1)
                m2, i2 = nm2, ni2
            e2 = jnp.exp(m2 - m1)
            p1 = 1.0 / (1.0 + e2)
            p2 = e2 * p1
            zero = jnp.zeros((L,), jnp.float32)
            for e in range(NUM_EXPERTS):
                o_refs[e][sl] = jnp.where(i1 == e, p1,
                                          jnp.where(i2 == e, p2, zero))
            i1_ref[sl] = i1
            i2_ref[sl] = i2

        for e in range(NUM_EXPERTS):
            pltpu.sync_copy(o_refs[e], outt_hbm.at[e, pl.ds(base, bpw)])
        pltpu.sync_copy(i1_ref, idxt_hbm.at[0, pl.ds(base, bpw)])
        pltpu.sync_copy(i2_ref, idxt_hbm.at[1, pl.ds(base, bpw)])

    return route(lt)


@jax.jit
def kernel(x, W, b):
    wt = W.T
    b2 = b.reshape(NUM_EXPERTS, 1)
    lt = _tc_logits(x, wt, b2)
    outt, idxt = _sc_route(lt)
    return outt.T, idxt.T
